# baseline (speedup 1.0000x reference)
import math

import jax
import jax.numpy as jnp
from jax import lax
from jax.experimental import pallas as pl
from jax.experimental.pallas import tpu as pltpu

_LABEL_SMOOTHING = 0.1
_V = 100000
_B = 1024
_PAD = 0
_CONF = 1.0 - _LABEL_SMOOTHING
_S = _LABEL_SMOOTHING / (_V - 2)
_C_ROW = (_V - 2) * _S * math.log(_S) + _CONF * math.log(_CONF)

_BB = 256
_VB = 3200
_W = 4
_NVB = 8


def _tc_body(t_ref, *refs):
    x_refs, o_ref = refs[:_W], refs[_W]
    rb = pl.program_id(0)
    vb = pl.program_id(1)

    @pl.when((rb == 0) & (vb == 0))
    def _init():
        o_ref[...] = jnp.zeros_like(o_ref)

    t = t_ref[...]
    nonpad = (t != _PAD).astype(jnp.float32)
    t_eff = jnp.where(t != _PAD, t, -1)
    rowpart = jnp.zeros((_BB, 1), jnp.float32)
    tsum = jnp.float32(0.0)
    for w in range(_W):
        x = x_refs[w][...]
        cols = ((vb * _W + w) * _VB
                + lax.broadcasted_iota(jnp.int32, x.shape, 1))
        xm = jnp.where(cols < _V, x, 0.0) if w == _W - 1 else x
        rowpart = rowpart + jnp.sum(xm, axis=1, keepdims=True)
        tsum = tsum + jnp.sum(jnp.where(cols == t_eff, x, 0.0))
    contrib = -_S * jnp.sum(nonpad * rowpart) - (_CONF - _S) * tsum
    corr = jnp.sum(nonpad * (_C_ROW + _S * x_refs[0][:, 0:1]))
    contrib = contrib + jnp.where(vb == 0, corr, 0.0)
    o_ref[...] = o_ref[...] + contrib


def _tc_reduce(tgt2d, output):
    def _win(w):
        return pl.BlockSpec((_BB, _VB), lambda rb, vb, w=w: (rb, vb * _W + w))

    return pl.pallas_call(
        _tc_body,
        grid=(_B // _BB, _NVB),
        in_specs=[pl.BlockSpec((_BB, 1), lambda rb, vb: (rb, 0))]
                 + [_win(w) for w in range(_W)],
        out_specs=pl.BlockSpec((1, 1), lambda rb, vb: (0, 0)),
        out_shape=jax.ShapeDtypeStruct((1, 1), jnp.float32),
        compiler_params=pltpu.CompilerParams(
            dimension_semantics=("arbitrary", "arbitrary")),
    )(tgt2d, *([output] * _W))


def kernel(output, target, one_hot):
    del one_hot
    tgt = target.astype(jnp.int32)
    loss = _tc_reduce(tgt.reshape(_B, 1), output)
    return loss[0, 0]
